# baseline (speedup 1.0000x reference)
import functools

import jax
import jax.numpy as jnp
from jax import lax
from jax.experimental import pallas as pl
from jax.experimental.pallas import tpu as pltpu
from jax.experimental.pallas import tpu_sc as plsc

_MAX_STEPS = 1000
_BATCH = 16384
_D = 512
_TPAD = 1024

_B_TC = 4096
_B_SC = _BATCH - _B_TC
_TC_BLK = 512

_NC = 2
_NS = 16
_NW = _NC * _NS
_ROWS_PER_W = _B_SC // _NW
_CHUNK = 64
_K = _ROWS_PER_W // _CHUNK
_NBUF = 2


def _build_table():
    steps = jnp.arange(_TPAD, dtype=jnp.float32)[:, None]
    dims = jnp.arange(64, dtype=jnp.float32)[None, :]
    t = steps * 10.0 ** (dims * 4.0 / 63.0)
    return jnp.concatenate([jnp.sin(t), jnp.cos(t)], axis=1)


def _mlp_body(t_ref, w1_ref, b1_ref, w2_ref, b2_ref, o_ref, obf_ref):
    x = t_ref[...]
    h = jnp.dot(x, w1_ref[...], preferred_element_type=jnp.float32) + b1_ref[...]
    h = h * jax.nn.sigmoid(h)
    o = jnp.dot(h, w2_ref[...], preferred_element_type=jnp.float32) + b2_ref[...]
    o = o * jax.nn.sigmoid(o)
    o_ref[...] = o
    obf_ref[...] = o.astype(jnp.bfloat16)


def _tc_mlp(table, W1, b1, W2, b2):
    return pl.pallas_call(
        _mlp_body,
        out_shape=(
            jax.ShapeDtypeStruct((_TPAD, _D), jnp.float32),
            jax.ShapeDtypeStruct((_TPAD, _D), jnp.bfloat16),
        ),
    )(table, W1, b1.reshape(1, _D), W2, b2.reshape(1, _D))


def _onehot_body(idx_ref, tbl_ref, o_ref):
    idx = idx_ref[0, 0, :]
    iota = lax.broadcasted_iota(jnp.int32, (_TC_BLK, _TPAD), 1)
    oh = (idx.reshape(_TC_BLK, 1) == iota).astype(jnp.bfloat16)
    o_ref[...] = jnp.dot(oh, tbl_ref[...], preferred_element_type=jnp.float32)


def _tc_onehot(idx_tc, tbl_bf16):
    nb = _B_TC // _TC_BLK
    return pl.pallas_call(
        _onehot_body,
        grid=(nb,),
        in_specs=[
            pl.BlockSpec((1, 1, _TC_BLK), lambda i: (i, 0, 0)),
            pl.BlockSpec((_TPAD, _D), lambda i: (0, 0)),
        ],
        out_specs=pl.BlockSpec((_TC_BLK, _D), lambda i: (i, 0)),
        out_shape=jax.ShapeDtypeStruct((_B_TC, _D), jnp.float32),
    )(idx_tc.reshape(nb, 1, _TC_BLK), tbl_bf16)


def _gather_body(table_hbm, idx_hbm, out_hbm, idx_v,
                 rows0, rows1, gsem0, gsem1, wsem0, wsem1):
    wid = lax.axis_index("s") * _NC + lax.axis_index("c")
    base = wid * _ROWS_PER_W
    bufs = (rows0, rows1)
    gsems = (gsem0, gsem1)
    wsems = (wsem0, wsem1)
    pltpu.sync_copy(idx_hbm.at[pl.ds(base, _ROWS_PER_W)], idx_v)
    g = [None] * _NBUF
    w = [None] * _NBUF
    for j in range(_K):
        b = j % _NBUF
        if w[b] is not None:
            w[b].wait()
        g[b] = pltpu.async_copy(
            table_hbm.at[idx_v.at[pl.ds(j * _CHUNK, _CHUNK)]],
            bufs[b], gsems[b])
        if j >= _NBUF - 1:
            jj = j - (_NBUF - 1)
            bb = jj % _NBUF
            g[bb].wait()
            w[bb] = pltpu.async_copy(
                bufs[bb], out_hbm.at[pl.ds(base + jj * _CHUNK, _CHUNK)],
                wsems[bb])
    for jj in range(_K - (_NBUF - 1), _K):
        bb = jj % _NBUF
        g[bb].wait()
        w[bb] = pltpu.async_copy(
            bufs[bb], out_hbm.at[pl.ds(base + jj * _CHUNK, _CHUNK)],
            wsems[bb])
    for bb in range(_NBUF):
        if w[bb] is not None:
            w[bb].wait()


def _sc_gather(final_table, idx):
    mesh = plsc.VectorSubcoreMesh(core_axis_name="c", subcore_axis_name="s")
    k = functools.partial(
        pl.kernel,
        mesh=mesh,
        out_type=jax.ShapeDtypeStruct((_B_SC, _D), jnp.float32),
        scratch_types=[
            pltpu.VMEM((_ROWS_PER_W,), jnp.int32),
            pltpu.VMEM((_CHUNK, _D), jnp.float32),
            pltpu.VMEM((_CHUNK, _D), jnp.float32),
            pltpu.SemaphoreType.DMA,
            pltpu.SemaphoreType.DMA,
            pltpu.SemaphoreType.DMA,
            pltpu.SemaphoreType.DMA,
        ],
    )(_gather_body)
    return k(final_table, idx)


def kernel(diffusion_step, W1, b1, W2, b2):
    table = _build_table()
    final_f32, final_bf16 = _tc_mlp(table, W1, b1, W2, b2)
    idx = diffusion_step.astype(jnp.int32)
    sc_part = _sc_gather(final_f32, idx[:_B_SC])
    tc_part = _tc_onehot(idx[_B_SC:], final_bf16)
    return jnp.concatenate([sc_part, tc_part], axis=0)


# speedup vs baseline: 1.3485x; 1.3485x over previous
import functools

import jax
import jax.numpy as jnp
from jax import lax
from jax.experimental import pallas as pl
from jax.experimental.pallas import tpu as pltpu
from jax.experimental.pallas import tpu_sc as plsc

_MAX_STEPS = 1000
_BATCH = 16384
_D = 512
_TPAD = 1024

_NC = 2
_NS = 16
_NW = _NC * _NS
_ROWS_PER_W = _BATCH // _NW
_CHUNK = 64
_K = _ROWS_PER_W // _CHUNK


def _build_table():
    steps = jnp.arange(_TPAD, dtype=jnp.float32)[:, None]
    dims = jnp.arange(64, dtype=jnp.float32)[None, :]
    t = steps * 10.0 ** (dims * 4.0 / 63.0)
    return jnp.concatenate([jnp.sin(t), jnp.cos(t)], axis=1)


def _mlp_body(t_ref, w1_ref, b1_ref, w2_ref, b2_ref, o_ref):
    x = t_ref[...]
    h = jnp.dot(x, w1_ref[...], preferred_element_type=jnp.float32) + b1_ref[...]
    h = h * jax.nn.sigmoid(h)
    o = jnp.dot(h, w2_ref[...], preferred_element_type=jnp.float32) + b2_ref[...]
    o_ref[...] = o * jax.nn.sigmoid(o)


def _tc_mlp(table, W1, b1, W2, b2):
    return pl.pallas_call(
        _mlp_body,
        out_shape=jax.ShapeDtypeStruct((_TPAD, _D), jnp.float32),
    )(table, W1, b1.reshape(1, _D), W2, b2.reshape(1, _D))


def _gather_body(table_hbm, idx_hbm, out_hbm, idx_v,
                 rows0, rows1, gsem0, gsem1, wsem0, wsem1):
    wid = lax.axis_index("s") * _NC + lax.axis_index("c")
    base = wid * _ROWS_PER_W
    pltpu.sync_copy(idx_hbm.at[pl.ds(base, _ROWS_PER_W)], idx_v)

    def gath(c, buf, sem):
        return pltpu.async_copy(
            table_hbm.at[idx_v.at[pl.ds(c * _CHUNK, _CHUNK)]], buf, sem)

    def wr(c, buf, sem):
        return pltpu.async_copy(
            buf, out_hbm.at[pl.ds(base + c * _CHUNK, _CHUNK)], sem)

    def wait_g(c, buf, sem):
        pltpu.make_async_copy(
            table_hbm.at[idx_v.at[pl.ds(c * _CHUNK, _CHUNK)]], buf, sem).wait()

    def wait_w(c, buf, sem):
        pltpu.make_async_copy(
            buf, out_hbm.at[pl.ds(base + c * _CHUNK, _CHUNK)], sem).wait()

    gath(0, rows0, gsem0)
    gath(1, rows1, gsem1)

    def body(t, carry):
        c = 2 * t
        wait_g(c, rows0, gsem0)
        wr(c, rows0, wsem0)
        wait_g(c + 1, rows1, gsem1)
        wr(c + 1, rows1, wsem1)
        wait_w(c, rows0, wsem0)
        gath(c + 2, rows0, gsem0)
        wait_w(c + 1, rows1, wsem1)
        gath(c + 3, rows1, gsem1)
        return carry

    lax.fori_loop(0, _K // 2 - 1, body, 0)
    c = _K - 2
    wait_g(c, rows0, gsem0)
    wr(c, rows0, wsem0)
    wait_g(c + 1, rows1, gsem1)
    wr(c + 1, rows1, wsem1)
    wait_w(c, rows0, wsem0)
    wait_w(c + 1, rows1, wsem1)


def _sc_gather(final_table, idx):
    mesh = plsc.VectorSubcoreMesh(core_axis_name="c", subcore_axis_name="s")
    k = functools.partial(
        pl.kernel,
        mesh=mesh,
        out_type=jax.ShapeDtypeStruct((_BATCH, _D), jnp.float32),
        scratch_types=[
            pltpu.VMEM((_ROWS_PER_W,), jnp.int32),
            pltpu.VMEM((_CHUNK, _D), jnp.float32),
            pltpu.VMEM((_CHUNK, _D), jnp.float32),
            pltpu.SemaphoreType.DMA,
            pltpu.SemaphoreType.DMA,
            pltpu.SemaphoreType.DMA,
            pltpu.SemaphoreType.DMA,
        ],
    )(_gather_body)
    return k(final_table, idx)


def kernel(diffusion_step, W1, b1, W2, b2):
    table = _build_table()
    final_table = _tc_mlp(table, W1, b1, W2, b2)
    return _sc_gather(final_table, diffusion_step.astype(jnp.int32))


# speedup vs baseline: 1.3744x; 1.0192x over previous
import functools

import jax
import jax.numpy as jnp
from jax import lax
from jax.experimental import pallas as pl
from jax.experimental.pallas import tpu as pltpu
from jax.experimental.pallas import tpu_sc as plsc

_MAX_STEPS = 1000
_BATCH = 16384
_D = 512
_TPAD = 1024

_NC = 2
_NS = 16
_NW = _NC * _NS
_ROWS_PER_W = _BATCH // _NW
_CHUNK = 64
_K = _ROWS_PER_W // _CHUNK


def _build_table():
    steps = jnp.arange(_TPAD, dtype=jnp.float32)[:, None]
    dims = jnp.arange(64, dtype=jnp.float32)[None, :]
    t = steps * 10.0 ** (dims * 4.0 / 63.0)
    return jnp.concatenate([jnp.sin(t), jnp.cos(t)], axis=1)


def _mlp_body(t_ref, w1_ref, b1_ref, w2_ref, b2_ref, o_ref):
    x = t_ref[...]
    h = jnp.dot(x, w1_ref[...], preferred_element_type=jnp.float32) + b1_ref[...]
    h = h * jax.nn.sigmoid(h)
    o = jnp.dot(h, w2_ref[...], preferred_element_type=jnp.float32) + b2_ref[...]
    o_ref[...] = o * jax.nn.sigmoid(o)


def _tc_mlp(table, W1, b1, W2, b2):
    return pl.pallas_call(
        _mlp_body,
        out_shape=jax.ShapeDtypeStruct((_TPAD, _D), jnp.float32),
    )(table, W1, b1, W2, b2)


def _gather_body(table_hbm, idx_hbm, out_hbm, idx_v,
                 rows0, rows1, gsem0, gsem1, wsem0, wsem1):
    wid = lax.axis_index("s") * _NC + lax.axis_index("c")
    base = wid * _ROWS_PER_W
    pltpu.sync_copy(idx_hbm.at[pl.ds(base, _ROWS_PER_W)], idx_v)

    def gath(c, buf, sem):
        return pltpu.async_copy(
            table_hbm.at[idx_v.at[pl.ds(c * _CHUNK, _CHUNK)]], buf, sem)

    def wr(c, buf, sem):
        return pltpu.async_copy(
            buf, out_hbm.at[pl.ds(base + c * _CHUNK, _CHUNK)], sem)

    bufs = (rows0, rows1)
    gsems = (gsem0, gsem1)
    wsems = (wsem0, wsem1)
    g = [None, None]
    w = [None, None]
    for j in range(_K):
        b = j % 2
        if w[b] is not None:
            w[b].wait()
        g[b] = gath(j, bufs[b], gsems[b])
        if j >= 1:
            bb = (j - 1) % 2
            g[bb].wait()
            w[bb] = wr(j - 1, bufs[bb], wsems[bb])
    g[(_K - 1) % 2].wait()
    w[(_K - 1) % 2] = wr(_K - 1, bufs[(_K - 1) % 2], wsems[(_K - 1) % 2])
    w[0].wait()
    w[1].wait()


def _sc_gather(final_table, idx):
    mesh = plsc.VectorSubcoreMesh(core_axis_name="c", subcore_axis_name="s")
    k = functools.partial(
        pl.kernel,
        mesh=mesh,
        out_type=jax.ShapeDtypeStruct((_BATCH, _D), jnp.float32),
        scratch_types=[
            pltpu.VMEM((_ROWS_PER_W,), jnp.int32),
            pltpu.VMEM((_CHUNK, _D), jnp.float32),
            pltpu.VMEM((_CHUNK, _D), jnp.float32),
            pltpu.SemaphoreType.DMA,
            pltpu.SemaphoreType.DMA,
            pltpu.SemaphoreType.DMA,
            pltpu.SemaphoreType.DMA,
        ],
    )(_gather_body)
    return k(final_table, idx)


def kernel(diffusion_step, W1, b1, W2, b2):
    table = _build_table()
    final_table = _tc_mlp(table, W1, b1, W2, b2)
    return _sc_gather(final_table, diffusion_step.astype(jnp.int32))
